# baseline (speedup 1.0000x reference)
import functools

import numpy as np
import jax
import jax.numpy as jnp
from jax import lax
from jax.experimental import pallas as pl
from jax.experimental.pallas import tpu as pltpu
from jax.experimental.pallas import tpu_sc as plsc

N_NODES = 10000
N_EDGES = 320000
DIM = 128
D_EDGE = 16

_kn, _ke = jax.random.split(jax.random.key(42))
_NODE_MASK = np.asarray(jax.random.bernoulli(_kn, 0.1, (N_NODES,)))
_KEEP = np.nonzero(np.asarray(jax.random.bernoulli(_ke, 0.9, (N_EDGES,))))[0]
_K = int(_KEEP.size)
_KEEP = _KEEP.astype(np.int32)
_KEEP_ROW1 = (_KEEP + np.int32(N_EDGES)).astype(np.int32)
_MASK_COL = _NODE_MASK.astype(np.float32).reshape(N_NODES, 1)

_NC, _NS = 2, 16
_NW = _NC * _NS
_BPW = -(-_K // _NW)
_BPW = -(-_BPW // 8) * 8
_L_LAST = _K - (_NW - 1) * _BPW
assert 0 < _L_LAST <= _BPW
_CH = 2048
_IN_S = 2560


def _chunks(total, step):
    out = [step] * (total // step)
    if total % step:
        out.append(total % step)
    return out


def _sc_body(keep, keep1, attr, eidx, attr_out, row0_out, row1_out,
             kva, kvb, ev, sa, oa, sem):
    c = lax.axis_index("c")
    s = lax.axis_index("s")
    w = s * _NC + c
    lane = jnp.arange(16, dtype=jnp.int32)

    def work(base, length):
        pltpu.sync_copy(keep.at[pl.ds(base, length)], kva.at[pl.ds(0, length)])
        pltpu.sync_copy(keep1.at[pl.ds(base, length)], kvb.at[pl.ds(0, length)])
        off = 0
        for n in _chunks(length, _CH):
            start = kva[pl.ds(off, 16)][0]
            in_off = jnp.minimum(start & ~jnp.int32(7),
                                 jnp.int32(N_EDGES - _IN_S))
            in_off = pl.multiple_of(in_off, 8)
            pltpu.sync_copy(attr.at[pl.ds(in_off * D_EDGE, _IN_S * D_EDGE)], sa)

            def grp(g, _):
                kc = kva[pl.ds(off + g * 16, 16)]
                obase = g * (16 * D_EDGE)
                for j in range(16):
                    li = jnp.clip(kc[j] - in_off, 0, _IN_S - 1)
                    oa[pl.ds(obase + j * D_EDGE, D_EDGE)] = (
                        sa[pl.ds(li * D_EDGE, D_EDGE)])
                return 0

            lax.fori_loop(0, -(-n // 16), grp, 0)
            pltpu.sync_copy(oa.at[pl.ds(0, n * D_EDGE)],
                            attr_out.at[pl.ds((base + off) * D_EDGE, n * D_EDGE)])
            off += n
        pltpu.async_copy(eidx.at[kva.at[pl.ds(0, length)]],
                         ev.at[pl.ds(0, length)], sem).wait()
        pltpu.sync_copy(ev.at[pl.ds(0, length)], row0_out.at[pl.ds(base, length)])
        pltpu.async_copy(eidx.at[kvb.at[pl.ds(0, length)]],
                         ev.at[pl.ds(0, length)], sem).wait()
        pltpu.sync_copy(ev.at[pl.ds(0, length)], row1_out.at[pl.ds(base, length)])

    @pl.when(w < _NW - 1)
    def _():
        work(w * _BPW, _BPW)

    @pl.when(w == _NW - 1)
    def _():
        work((_NW - 1) * _BPW, _L_LAST)


@functools.partial(jax.jit, static_argnums=())
def _sc_gather(keep, keep1, attr, eidx_flat):
    fn = pl.kernel(
        _sc_body,
        out_type=[
            jax.ShapeDtypeStruct((_K * D_EDGE,), jnp.float32),
            jax.ShapeDtypeStruct((_K,), jnp.int32),
            jax.ShapeDtypeStruct((_K,), jnp.int32),
        ],
        mesh=plsc.VectorSubcoreMesh(core_axis_name="c", subcore_axis_name="s",
                                    num_cores=_NC, num_subcores=_NS),
        scratch_types=[
            pltpu.VMEM((_BPW + 16,), jnp.int32),
            pltpu.VMEM((_BPW,), jnp.int32),
            pltpu.VMEM((_BPW,), jnp.int32),
            pltpu.VMEM((_IN_S * D_EDGE,), jnp.float32),
            pltpu.VMEM((_CH * D_EDGE,), jnp.float32),
            pltpu.SemaphoreType.DMA,
        ],
    )
    return fn(keep, keep1, attr, eidx_flat)


def _tc_where_body(m_ref, tok_ref, x_ref, o_ref):
    o_ref[...] = jnp.where(m_ref[...] > 0.5, tok_ref[...], x_ref[...])


def _tc_where(x, token):
    blk = 1000
    return pl.pallas_call(
        _tc_where_body,
        grid=(N_NODES // blk,),
        in_specs=[
            pl.BlockSpec((blk, 1), lambda i: (i, 0)),
            pl.BlockSpec((1, DIM), lambda i: (0, 0)),
            pl.BlockSpec((blk, DIM), lambda i: (i, 0)),
        ],
        out_specs=pl.BlockSpec((blk, DIM), lambda i: (i, 0)),
        out_shape=jax.ShapeDtypeStruct((N_NODES, DIM), jnp.float32),
    )(jnp.asarray(_MASK_COL), token.reshape(1, DIM), x)


def kernel(x, edge_idx, edge_attr, token):
    x_out = _tc_where(x, token)
    attr_out, row0, row1 = _sc_gather(
        jnp.asarray(_KEEP), jnp.asarray(_KEEP_ROW1),
        edge_attr.reshape(-1), edge_idx.reshape(-1))
    return x_out, jnp.stack([row0, row1]), attr_out.reshape(_K, D_EDGE)


# speedup vs baseline: 1.0898x; 1.0898x over previous
import functools

import numpy as np
import jax
import jax.numpy as jnp
from jax import lax
from jax.experimental import pallas as pl
from jax.experimental.pallas import tpu as pltpu
from jax.experimental.pallas import tpu_sc as plsc

N_NODES = 10000
N_EDGES = 320000
DIM = 128
D_EDGE = 16

_kn, _ke = jax.random.split(jax.random.key(42))
_NODE_MASK = np.asarray(jax.random.bernoulli(_kn, 0.1, (N_NODES,)))
_KEEP = np.nonzero(np.asarray(jax.random.bernoulli(_ke, 0.9, (N_EDGES,))))[0]
_K = int(_KEEP.size)
_KEEP = _KEEP.astype(np.int32)
_MASK_COL = _NODE_MASK.astype(np.float32).reshape(N_NODES, 1)

_KP = -(-_K // 128) * 128
_KEEP_PAD = np.concatenate([_KEEP, np.full(_KP - _K, _KEEP[-1], np.int32)])
_NC, _NS = 2, 16
_NW = _NC * _NS
_BPW = 8960
_L_LAST = _KP - (_NW - 1) * _BPW
_L_LAST_ATTR = _K - (_NW - 1) * _BPW
assert 0 < _L_LAST_ATTR <= _L_LAST
_CH = 2048
_IN_S = 2560


def _chunks(total, step):
    out = [step] * (total // step)
    if total % step:
        out.append(total % step)
    return out


def _sc_body(keep, attr, eidx, attr_out, idx_out,
             kva, sa, oa, s0, s1, ob0, ob1):
    c = lax.axis_index("c")
    s = lax.axis_index("s")
    w = s * _NC + c

    def work(base, length, length_attr):
        pltpu.sync_copy(keep.at[pl.ds(base, length)], kva.at[pl.ds(0, length)])
        off = 0
        for n in _chunks(length, _CH):
            n_attr = max(0, min(length_attr - off, n))
            start = kva[pl.ds(off, 16)][0]
            in_off = jnp.minimum(start & ~jnp.int32(127),
                                 jnp.int32(N_EDGES - _IN_S))
            in_off = pl.multiple_of(in_off, 128)
            pltpu.sync_copy(attr.at[pl.ds(in_off * D_EDGE, _IN_S * D_EDGE)], sa)
            pltpu.sync_copy(eidx.at[0, pl.ds(in_off, _IN_S)], s0.at[pl.ds(0, _IN_S)])
            pltpu.sync_copy(eidx.at[1, pl.ds(in_off, _IN_S)], s1.at[pl.ds(0, _IN_S)])

            def grp(g, _):
                kc = kva[pl.ds(off + g * 16, 16)]
                li = jnp.clip(kc - in_off, 0, _IN_S - 1)
                w0 = li[0]
                rel = li - w0
                lo = jnp.minimum(rel, 15)
                hi = jnp.clip(rel - 16, 0, 15)
                sel = rel < 16
                for src, dst in ((s0, ob0), (s1, ob1)):
                    v0 = src[pl.ds(w0, 16)]
                    v1 = src[pl.ds(w0 + 16, 16)]
                    dst[pl.ds(g * 16, 16)] = jnp.where(
                        sel,
                        v0.at[lo].get(mode="promise_in_bounds"),
                        v1.at[hi].get(mode="promise_in_bounds"))
                obase = g * (16 * D_EDGE)
                for j in range(16):
                    oa[pl.ds(obase + j * D_EDGE, D_EDGE)] = (
                        sa[pl.ds(li[j] * D_EDGE, D_EDGE)])
                return 0

            lax.fori_loop(0, -(-n // 16), grp, 0)
            if n_attr > 0:
                pltpu.sync_copy(
                    oa.at[pl.ds(0, n_attr * D_EDGE)],
                    attr_out.at[pl.ds((base + off) * D_EDGE, n_attr * D_EDGE)])
            pltpu.sync_copy(ob0.at[pl.ds(0, n)], idx_out.at[0, pl.ds(base + off, n)])
            pltpu.sync_copy(ob1.at[pl.ds(0, n)], idx_out.at[1, pl.ds(base + off, n)])
            off += n

    @pl.when(w < _NW - 1)
    def _():
        work(w * _BPW, _BPW, _BPW)

    @pl.when(w == _NW - 1)
    def _():
        work((_NW - 1) * _BPW, _L_LAST, _L_LAST_ATTR)


@jax.jit
def _sc_gather(keep, attr_flat, eidx):
    fn = pl.kernel(
        _sc_body,
        out_type=[
            jax.ShapeDtypeStruct((_K * D_EDGE,), jnp.float32),
            jax.ShapeDtypeStruct((2, _KP), jnp.int32),
        ],
        mesh=plsc.VectorSubcoreMesh(core_axis_name="c", subcore_axis_name="s",
                                    num_cores=_NC, num_subcores=_NS),
        scratch_types=[
            pltpu.VMEM((_L_LAST + 16,), jnp.int32),
            pltpu.VMEM((_IN_S * D_EDGE,), jnp.float32),
            pltpu.VMEM((_CH * D_EDGE,), jnp.float32),
            pltpu.VMEM((_IN_S + 32,), jnp.int32),
            pltpu.VMEM((_IN_S + 32,), jnp.int32),
            pltpu.VMEM((_CH,), jnp.int32),
            pltpu.VMEM((_CH,), jnp.int32),
        ],
    )
    return fn(keep, attr_flat, eidx)


def _tc_where_body(m_ref, tok_ref, x_ref, o_ref):
    o_ref[...] = jnp.where(m_ref[...] > 0.5, tok_ref[...], x_ref[...])


def _tc_where(x, token):
    blk = 1000
    return pl.pallas_call(
        _tc_where_body,
        grid=(N_NODES // blk,),
        in_specs=[
            pl.BlockSpec((blk, 1), lambda i: (i, 0)),
            pl.BlockSpec((1, DIM), lambda i: (0, 0)),
            pl.BlockSpec((blk, DIM), lambda i: (i, 0)),
        ],
        out_specs=pl.BlockSpec((blk, DIM), lambda i: (i, 0)),
        out_shape=jax.ShapeDtypeStruct((N_NODES, DIM), jnp.float32),
    )(jnp.asarray(_MASK_COL), token.reshape(1, DIM), x)


def kernel(x, edge_idx, edge_attr, token):
    x_out = _tc_where(x, token)
    attr_out, idx_out = _sc_gather(
        jnp.asarray(_KEEP_PAD), edge_attr.reshape(-1), edge_idx)
    return x_out, idx_out[:, :_K], attr_out.reshape(_K, D_EDGE)


# speedup vs baseline: 1.0993x; 1.0087x over previous
import functools

import numpy as np
import jax
import jax.numpy as jnp
from jax import lax
from jax.experimental import pallas as pl
from jax.experimental.pallas import tpu as pltpu
from jax.experimental.pallas import tpu_sc as plsc

N_NODES = 10000
N_EDGES = 320000
DIM = 128
D_EDGE = 16

_kn, _ke = jax.random.split(jax.random.key(42))
_NODE_MASK = np.asarray(jax.random.bernoulli(_kn, 0.1, (N_NODES,)))
_KEEP = np.nonzero(np.asarray(jax.random.bernoulli(_ke, 0.9, (N_EDGES,))))[0]
_K = int(_KEEP.size)
_KEEP = _KEEP.astype(np.int32)
_MASK_COL = _NODE_MASK.astype(np.float32).reshape(N_NODES, 1)

_KP = -(-_K // 128) * 128
_KEEP_PAD = np.concatenate([_KEEP, np.full(_KP - _K, _KEEP[-1], np.int32)])
_NC, _NS = 2, 16
_NW = _NC * _NS
_BPW = 8960
_L_LAST = _KP - (_NW - 1) * _BPW
_L_LAST_ATTR = _K - (_NW - 1) * _BPW
assert 0 < _L_LAST_ATTR <= _L_LAST
_CH = 2048
_IN_S = 2560


def _chunks(total, step):
    out = [step] * (total // step)
    if total % step:
        out.append(total % step)
    return out


def _sc_body(keep, attr, eidx, attr_out, r0_out, r1_out,
             kva, sa, oa, s0, s1, ob0, ob1):
    c = lax.axis_index("c")
    s = lax.axis_index("s")
    w = s * _NC + c

    def work(base, length, length_attr):
        pltpu.sync_copy(keep.at[pl.ds(base, length)], kva.at[pl.ds(0, length)])
        off = 0
        for n in _chunks(length, _CH):
            n_attr = max(0, min(length_attr - off, n))
            start = kva[pl.ds(off, 16)][0]
            in_off = jnp.minimum(start & ~jnp.int32(127),
                                 jnp.int32(N_EDGES - _IN_S))
            in_off = pl.multiple_of(in_off, 128)
            pltpu.sync_copy(attr.at[pl.ds(in_off * D_EDGE, _IN_S * D_EDGE)], sa)
            pltpu.sync_copy(eidx.at[0, pl.ds(in_off, _IN_S)], s0.at[pl.ds(0, _IN_S)])
            pltpu.sync_copy(eidx.at[1, pl.ds(in_off, _IN_S)], s1.at[pl.ds(0, _IN_S)])

            def grp(g, _):
                kc = kva[pl.ds(off + g * 16, 16)]
                li = jnp.clip(kc - in_off, 0, _IN_S - 1)
                w0 = li[0]
                rel = li - w0
                lo = jnp.minimum(rel, 15)
                hi = jnp.clip(rel - 16, 0, 15)
                sel = rel < 16
                for src, dst in ((s0, ob0), (s1, ob1)):
                    v0 = src[pl.ds(w0, 16)]
                    v1 = src[pl.ds(w0 + 16, 16)]
                    dst[pl.ds(g * 16, 16)] = jnp.where(
                        sel,
                        v0.at[lo].get(mode="promise_in_bounds"),
                        v1.at[hi].get(mode="promise_in_bounds"))
                obase = g * (16 * D_EDGE)
                for j in range(16):
                    oa[pl.ds(obase + j * D_EDGE, D_EDGE)] = (
                        sa[pl.ds(li[j] * D_EDGE, D_EDGE)])
                return 0

            lax.fori_loop(0, -(-n // 16), grp, 0)
            if n_attr > 0:
                pltpu.sync_copy(
                    oa.at[pl.ds(0, n_attr * D_EDGE)],
                    attr_out.at[pl.ds((base + off) * D_EDGE, n_attr * D_EDGE)])
            pltpu.sync_copy(ob0.at[pl.ds(0, n)], r0_out.at[pl.ds(base + off, n)])
            pltpu.sync_copy(ob1.at[pl.ds(0, n)], r1_out.at[pl.ds(base + off, n)])
            off += n

    @pl.when(w < _NW - 1)
    def _():
        work(w * _BPW, _BPW, _BPW)

    @pl.when(w == _NW - 1)
    def _():
        work((_NW - 1) * _BPW, _L_LAST, _L_LAST_ATTR)


@jax.jit
def _sc_gather(keep, attr_flat, eidx):
    fn = pl.kernel(
        _sc_body,
        out_type=[
            jax.ShapeDtypeStruct((_K * D_EDGE,), jnp.float32),
            jax.ShapeDtypeStruct((_KP,), jnp.int32),
            jax.ShapeDtypeStruct((_KP,), jnp.int32),
        ],
        mesh=plsc.VectorSubcoreMesh(core_axis_name="c", subcore_axis_name="s",
                                    num_cores=_NC, num_subcores=_NS),
        scratch_types=[
            pltpu.VMEM((_L_LAST + 16,), jnp.int32),
            pltpu.VMEM((_IN_S * D_EDGE,), jnp.float32),
            pltpu.VMEM((_CH * D_EDGE,), jnp.float32),
            pltpu.VMEM((_IN_S + 32,), jnp.int32),
            pltpu.VMEM((_IN_S + 32,), jnp.int32),
            pltpu.VMEM((_CH,), jnp.int32),
            pltpu.VMEM((_CH,), jnp.int32),
        ],
    )
    return fn(keep, attr_flat, eidx)


_BKI = 32768


def _fin_idx_body(r0_ref, r1_ref, o_ref):
    o_ref[0:1, :] = r0_ref[...].reshape(1, _BKI)
    o_ref[1:2, :] = r1_ref[...].reshape(1, _BKI)


def _fin_idx(r0, r1):
    return pl.pallas_call(
        _fin_idx_body,
        grid=(-(-_K // _BKI),),
        in_specs=[pl.BlockSpec((_BKI,), lambda i: (i,)),
                  pl.BlockSpec((_BKI,), lambda i: (i,))],
        out_specs=pl.BlockSpec((2, _BKI), lambda i: (0, i)),
        out_shape=jax.ShapeDtypeStruct((2, _K), jnp.int32),
    )(r0, r1)


def _tc_where_body(m_ref, tok_ref, x_ref, o_ref):
    o_ref[...] = jnp.where(m_ref[...] > 0.5, tok_ref[...], x_ref[...])


def _tc_where(x, token):
    blk = 1000
    return pl.pallas_call(
        _tc_where_body,
        grid=(N_NODES // blk,),
        in_specs=[
            pl.BlockSpec((blk, 1), lambda i: (i, 0)),
            pl.BlockSpec((1, DIM), lambda i: (0, 0)),
            pl.BlockSpec((blk, DIM), lambda i: (i, 0)),
        ],
        out_specs=pl.BlockSpec((blk, DIM), lambda i: (i, 0)),
        out_shape=jax.ShapeDtypeStruct((N_NODES, DIM), jnp.float32),
    )(jnp.asarray(_MASK_COL), token.reshape(1, DIM), x)


def kernel(x, edge_idx, edge_attr, token):
    x_out = _tc_where(x, token)
    attr_flat, r0, r1 = _sc_gather(
        jnp.asarray(_KEEP_PAD), edge_attr.reshape(-1), edge_idx)
    return x_out, _fin_idx(r0, r1), attr_flat.reshape(_K, D_EDGE)
